# Initial kernel scaffold; baseline (speedup 1.0000x reference)
import jax
import jax.numpy as jnp
from jax.experimental import pallas as pl


def kernel(x, t, loc_w0, loc_w1, loc_w2, time_w0, time_w1):
    raise NotImplementedError("write your pallas kernel here")


# speedup vs baseline: 5.9343x; 5.9343x over previous
import jax
import jax.numpy as jnp
from jax import lax
from jax.experimental import pallas as pl
from jax.experimental.pallas import tpu as pltpu
from jax.experimental.pallas import tpu_sc as plsc

_B, _L = 16384, 20
_BL = _B * _L
_DLOC, _DTIME = 64, 32
_DOUT = _DLOC + _DTIME
_NC, _NS = 2, 16
_NW = _NC * _NS
_C = 128
_TOK_PER_W = _BL // _NW
_CHUNKS_PER_W = _TOK_PER_W // _C
_G = _BL // _C


def _body(idx_hbm, loc0, loc1, loc2, tw0, tw1, out_hbm,
          idx_v, b0, b1, b2, tb0, tb1, out_v, sem):
    wid = lax.axis_index("s") * _NC + lax.axis_index("c")

    def chunk(g, _):
        gid = wid * _CHUNKS_PER_W + g
        base = gid * _C
        pltpu.sync_copy(idx_hbm.at[gid], idx_v)
        d0 = pltpu.async_copy(loc0.at[idx_v.at[0]], b0, sem)
        d1 = pltpu.async_copy(loc1.at[idx_v.at[1]], b1, sem)
        d2 = pltpu.async_copy(loc2.at[idx_v.at[2]], b2, sem)
        d3 = pltpu.async_copy(tw0.at[idx_v.at[3]], tb0, sem)
        d4 = pltpu.async_copy(tw1.at[idx_v.at[4]], tb1, sem)
        d0.wait(); d1.wait(); d2.wait(); d3.wait(); d4.wait()

        def row(i, _):
            for j in range(_DLOC // 16):
                s = pl.ds(j * 16, 16)
                out_v[i, s] = b0[i, s] + b1[i, s] + b2[i, s]
            for j in range(_DTIME // 16):
                s = pl.ds(j * 16, 16)
                out_v[i, pl.ds(_DLOC + j * 16, 16)] = tb0[i, s] + tb1[i, s]
            return ()

        lax.fori_loop(0, _C, row, ())
        pltpu.sync_copy(out_v, out_hbm.at[pl.ds(base, _C)])
        return ()

    lax.fori_loop(0, _CHUNKS_PER_W, chunk, ())


def kernel(x, t, loc_w0, loc_w1, loc_w2, time_w0, time_w1):
    xf = x.reshape(_BL, 3).astype(jnp.int32)
    tf = t.reshape(_BL, 2).astype(jnp.int32)
    idx_all = jnp.stack(
        [xf[:, 0].reshape(_G, _C), xf[:, 1].reshape(_G, _C),
         xf[:, 2].reshape(_G, _C), tf[:, 0].reshape(_G, _C),
         tf[:, 1].reshape(_G, _C)], axis=1)

    mesh = plsc.VectorSubcoreMesh(core_axis_name="c", subcore_axis_name="s",
                                  num_cores=_NC, num_subcores=_NS)
    out = pl.kernel(
        _body,
        out_type=jax.ShapeDtypeStruct((_BL, _DOUT), jnp.float32),
        mesh=mesh,
        scratch_types=[
            pltpu.VMEM((5, _C), jnp.int32),
            pltpu.VMEM((_C, _DLOC), jnp.float32),
            pltpu.VMEM((_C, _DLOC), jnp.float32),
            pltpu.VMEM((_C, _DLOC), jnp.float32),
            pltpu.VMEM((_C, _DTIME), jnp.float32),
            pltpu.VMEM((_C, _DTIME), jnp.float32),
            pltpu.VMEM((_C, _DOUT), jnp.float32),
            pltpu.SemaphoreType.DMA,
        ],
        compiler_params=pltpu.CompilerParams(use_tc_tiling_on_sc=False),
    )(idx_all, loc_w0, loc_w1, loc_w2, time_w0, time_w1)
    return out.reshape(_B, _L, _DOUT)


# speedup vs baseline: 7.2843x; 1.2275x over previous
import jax
import jax.numpy as jnp
from jax import lax
from jax.experimental import pallas as pl
from jax.experimental.pallas import tpu as pltpu
from jax.experimental.pallas import tpu_sc as plsc

_B, _L = 16384, 20
_BL = _B * _L
_DLOC, _DTIME = 64, 32
_DOUT = _DLOC + _DTIME
_NC, _NS = 2, 16
_NW = _NC * _NS
_C = 128
_MB = 16
_TOK_PER_W = _BL // _NW
_CHUNKS_PER_W = _TOK_PER_W // _C
_NMB = _CHUNKS_PER_W // _MB
_G = _BL // _C


def _body(idx_hbm, loc0, loc1, loc2, tw0, tw1, out_hbm,
          idx_v, b0, b1, b2, tb0, tb1, out_v, sg0, sg1, ss0, ss1):
    wid = lax.axis_index("s") * _NC + lax.axis_index("c")
    sg = (sg0, sg1)
    ss = (ss0, ss1)

    def gather_descs(slot, lc):
        s = sg[slot]
        return (
            pltpu.make_async_copy(loc0.at[idx_v.at[lc, 0]], b0.at[slot], s),
            pltpu.make_async_copy(loc1.at[idx_v.at[lc, 1]], b1.at[slot], s),
            pltpu.make_async_copy(loc2.at[idx_v.at[lc, 2]], b2.at[slot], s),
            pltpu.make_async_copy(tw0.at[idx_v.at[lc, 3]], tb0.at[slot], s),
            pltpu.make_async_copy(tw1.at[idx_v.at[lc, 4]], tb1.at[slot], s),
        )

    def fire(slot, lc):
        for d in gather_descs(slot, lc):
            d.start()

    def wait_gathers(slot, lc):
        for d in gather_descs(slot, lc):
            d.wait()

    def scatter_desc(slot, gg):
        return pltpu.make_async_copy(
            out_v.at[slot], out_hbm.at[pl.ds(gg * _C, _C)], ss[slot])

    def compute(slot):
        def row(i, _):
            for j in range(_DLOC // 16):
                s = pl.ds(j * 16, 16)
                out_v[slot, i, s] = (b0[slot, i, s] + b1[slot, i, s]
                                     + b2[slot, i, s])
            for j in range(_DTIME // 16):
                s = pl.ds(j * 16, 16)
                out_v[slot, i, pl.ds(_DLOC + j * 16, 16)] = (
                    tb0[slot, i, s] + tb1[slot, i, s])
            return ()
        lax.fori_loop(0, _C, row, ())

    @pl.loop(0, _NMB)
    def megablock(mb):
        mb_base = wid * _CHUNKS_PER_W + mb * _MB
        pltpu.sync_copy(idx_hbm.at[pl.ds(mb_base, _MB)], idx_v)
        fire(0, 0)

        @pl.loop(0, _MB, step=2)
        def pair(k):
            for b in range(2):
                lc = k + b
                nxt = lc + 1

                @pl.when(nxt < _MB)
                def _():
                    fire(1 - b, nxt)

                wait_gathers(b, lc)
                compute(b)
                gg = mb_base + lc

                @pl.when(mb * _MB + lc >= 2)
                def _():
                    scatter_desc(b, gg).wait()

                scatter_desc(b, gg).start()

    scatter_desc(0, wid * _CHUNKS_PER_W).wait()
    scatter_desc(1, wid * _CHUNKS_PER_W).wait()


def kernel(x, t, loc_w0, loc_w1, loc_w2, time_w0, time_w1):
    xf = x.reshape(_BL, 3).astype(jnp.int32)
    tf = t.reshape(_BL, 2).astype(jnp.int32)
    idx_all = jnp.stack(
        [xf[:, 0].reshape(_G, _C), xf[:, 1].reshape(_G, _C),
         xf[:, 2].reshape(_G, _C), tf[:, 0].reshape(_G, _C),
         tf[:, 1].reshape(_G, _C)], axis=1)

    mesh = plsc.VectorSubcoreMesh(core_axis_name="c", subcore_axis_name="s",
                                  num_cores=_NC, num_subcores=_NS)
    out = pl.kernel(
        _body,
        out_type=jax.ShapeDtypeStruct((_BL, _DOUT), jnp.float32),
        mesh=mesh,
        scratch_types=[
            pltpu.VMEM((_MB, 5, _C), jnp.int32),
            pltpu.VMEM((2, _C, _DLOC), jnp.float32),
            pltpu.VMEM((2, _C, _DLOC), jnp.float32),
            pltpu.VMEM((2, _C, _DLOC), jnp.float32),
            pltpu.VMEM((2, _C, _DTIME), jnp.float32),
            pltpu.VMEM((2, _C, _DTIME), jnp.float32),
            pltpu.VMEM((2, _C, _DOUT), jnp.float32),
            pltpu.SemaphoreType.DMA,
            pltpu.SemaphoreType.DMA,
            pltpu.SemaphoreType.DMA,
            pltpu.SemaphoreType.DMA,
        ],
        compiler_params=pltpu.CompilerParams(use_tc_tiling_on_sc=False),
    )(idx_all, loc_w0, loc_w1, loc_w2, time_w0, time_w1)
    return out.reshape(_B, _L, _DOUT)
